# Initial kernel scaffold; baseline (speedup 1.0000x reference)
import jax
import jax.numpy as jnp
from jax.experimental import pallas as pl


def kernel(x, table):
    raise NotImplementedError("write your pallas kernel here")


# speedup vs baseline: 1.8993x; 1.8993x over previous
import functools

import jax
import jax.numpy as jnp
from jax import lax
from jax.experimental import pallas as pl
from jax.experimental.pallas import tpu as pltpu
from jax.experimental.pallas import tpu_sc as plsc

VOCAB = 128
DIM = 128
BATCH = 16384


@functools.cache
def _build():
    info = plsc.get_sparse_core_info()
    nc, ns = info.num_cores, info.num_subcores
    nw = nc * ns
    b_per_w = BATCH // nw

    mesh = plsc.VectorSubcoreMesh(core_axis_name="c", subcore_axis_name="s")

    @functools.partial(
        pl.kernel,
        mesh=mesh,
        out_type=jax.ShapeDtypeStruct((BATCH, DIM), jnp.float32),
        scratch_types=[
            pltpu.VMEM((b_per_w,), jnp.int32),
            pltpu.VMEM((b_per_w, DIM), jnp.float32),
            pltpu.SemaphoreType.DMA,
        ],
    )
    def gather_kernel(x_hbm, table_hbm, out_hbm, idx_v, rows_v, sem):
        wid = lax.axis_index("s") * nc + lax.axis_index("c")
        base = wid * b_per_w
        pltpu.sync_copy(x_hbm.at[pl.ds(base, b_per_w)], idx_v)
        pltpu.async_copy(table_hbm.at[idx_v], rows_v, sem).wait()
        pltpu.sync_copy(rows_v, out_hbm.at[pl.ds(base, b_per_w)])

    return gather_kernel


def kernel(x, table):
    return _build()(x.astype(jnp.int32), table)


# speedup vs baseline: 2.7453x; 1.4454x over previous
import functools

import jax
import jax.numpy as jnp
from jax import lax
from jax.experimental import pallas as pl
from jax.experimental.pallas import tpu as pltpu
from jax.experimental.pallas import tpu_sc as plsc

VOCAB = 128
DIM = 128
BATCH = 16384


@functools.cache
def _build():
    info = plsc.get_sparse_core_info()
    nc, ns = info.num_cores, info.num_subcores
    nw = nc * ns
    b_per_w = BATCH // nw

    mesh = plsc.VectorSubcoreMesh(core_axis_name="c", subcore_axis_name="s")

    @functools.partial(
        pl.kernel,
        mesh=mesh,
        out_type=jax.ShapeDtypeStruct((BATCH, DIM), jnp.float32),
        scratch_types=[
            pltpu.VMEM((b_per_w,), jnp.int32),
            pltpu.VMEM((b_per_w, DIM), jnp.float32),
            pltpu.VMEM_SHARED((VOCAB, DIM), jnp.float32),
            pltpu.SemaphoreType.DMA,
            pltpu.SemaphoreType.DMA,
        ],
    )
    def gather_kernel(x_hbm, table_hbm, out_hbm, idx_v, rows_v, table_sh, sem, tsem):
        s = lax.axis_index("s")
        wid = s * nc + lax.axis_index("c")
        base = wid * b_per_w
        tcopy = pltpu.make_async_copy(table_hbm, table_sh, tsem)

        @pl.when(s == 0)
        def _():
            tcopy.start()

        pltpu.sync_copy(x_hbm.at[pl.ds(base, b_per_w)], idx_v)

        @pl.when(s == 0)
        def _():
            tcopy.wait()

        plsc.subcore_barrier()
        pltpu.async_copy(table_sh.at[idx_v], rows_v, sem).wait()
        pltpu.sync_copy(rows_v, out_hbm.at[pl.ds(base, b_per_w)])

    return gather_kernel


def kernel(x, table):
    return _build()(x.astype(jnp.int32), table)


# speedup vs baseline: 2.8083x; 1.0230x over previous
import functools

import jax
import jax.numpy as jnp
from jax import lax
from jax.experimental import pallas as pl
from jax.experimental.pallas import tpu as pltpu
from jax.experimental.pallas import tpu_sc as plsc

VOCAB = 128
DIM = 128
BATCH = 16384


@functools.cache
def _build():
    info = plsc.get_sparse_core_info()
    nc, ns = info.num_cores, info.num_subcores
    nw = nc * ns
    b_per_w = BATCH // nw

    mesh = plsc.VectorSubcoreMesh(core_axis_name="c", subcore_axis_name="s")

    chunk = 128
    nchunk = b_per_w // chunk

    @functools.partial(
        pl.kernel,
        mesh=mesh,
        out_type=jax.ShapeDtypeStruct((BATCH, DIM), jnp.float32),
        scratch_types=[
            pltpu.VMEM((b_per_w,), jnp.int32),
            pltpu.VMEM((chunk, DIM), jnp.float32),
            pltpu.VMEM((chunk, DIM), jnp.float32),
            pltpu.VMEM_SHARED((VOCAB, DIM), jnp.float32),
            pltpu.SemaphoreType.DMA,
            pltpu.SemaphoreType.DMA,
            pltpu.SemaphoreType.DMA,
            pltpu.SemaphoreType.DMA,
            pltpu.SemaphoreType.DMA,
        ],
    )
    def gather_kernel(x_hbm, table_hbm, out_hbm, idx_v, rows0, rows1,
                      table_sh, g0, g1, w0, w1, tsem):
        s = lax.axis_index("s")
        wid = s * nc + lax.axis_index("c")
        base = wid * b_per_w
        bufs = (rows0, rows1)
        gsems = (g0, g1)
        wsems = (w0, w1)
        tcopy = pltpu.make_async_copy(table_hbm, table_sh, tsem)

        @pl.when(s == 0)
        def _():
            tcopy.start()

        pltpu.sync_copy(x_hbm.at[pl.ds(base, b_per_w)], idx_v)

        @pl.when(s == 0)
        def _():
            tcopy.wait()

        plsc.subcore_barrier()

        gcs = []
        wcs = []
        for k in range(nchunk):
            b = k % 2
            if k >= 2:
                wcs[k - 2].wait()
            gc = pltpu.make_async_copy(
                table_sh.at[idx_v.at[pl.ds(k * chunk, chunk)]], bufs[b], gsems[b])
            gc.start()
            gcs.append(gc)
            if k >= 1:
                gcs[k - 1].wait()
                wc = pltpu.make_async_copy(
                    bufs[(k - 1) % 2], out_hbm.at[pl.ds(base + (k - 1) * chunk, chunk)],
                    wsems[(k - 1) % 2])
                wc.start()
                wcs.append(wc)
        gcs[-1].wait()
        wc = pltpu.make_async_copy(
            bufs[(nchunk - 1) % 2],
            out_hbm.at[pl.ds(base + (nchunk - 1) * chunk, chunk)],
            wsems[(nchunk - 1) % 2])
        wc.start()
        wcs.append(wc)
        wcs[-2].wait()
        wcs[-1].wait()

    return gather_kernel


def kernel(x, table):
    return _build()(x.astype(jnp.int32), table)
